# baseline (speedup 1.0000x reference)
import functools

import jax
import jax.numpy as jnp
from jax import lax
from jax.experimental import pallas as pl
from jax.experimental.pallas import tpu as pltpu
from jax.experimental.pallas import tpu_sc as plsc

N = 10000
E = 320000
D = 128
G = 64

NC = 2
NS = 16
CHUNK = 128
KCNT = 80
TOTCH = NC * NS * KCNT
E_PAD = TOTCH * CHUNK
KS = 40
ROWS_PER_TILE = 5 * CHUNK
N_ACC = NS * ROWS_PER_TILE
DUMMY = N

_MESH = plsc.VectorSubcoreMesh(core_axis_name="c", subcore_axis_name="s")


@functools.partial(
    pl.kernel,
    mesh=_MESH,
    out_type=jax.ShapeDtypeStruct((NC, N_ACC, D), jnp.float32),
    scratch_types=[
        pltpu.VMEM((KCNT, CHUNK), jnp.int32),
        pltpu.VMEM((CHUNK, D), jnp.float32),
        pltpu.VMEM_SHARED((N_ACC, D), jnp.float32),
    ],
)
def _sc_count(dst_hbm, out_hbm, dst_v, buf_v, acc_sh):
    c = lax.axis_index("c")
    s = lax.axis_index("s")
    row0 = s * ROWS_PER_TILE

    def fill(val):
        def body(i, _):
            for jj in range(D // 16):
                buf_v[i, pl.ds(jj * 16, 16)] = jnp.full((16,), val, jnp.float32)
            return 0
        lax.fori_loop(0, CHUNK, body, 0)

    fill(0.0)
    for r in range(ROWS_PER_TILE // CHUNK):
        pltpu.sync_copy(buf_v, acc_sh.at[pl.ds(row0 + r * CHUNK, CHUNK)])
    fill(1.0)
    pltpu.sync_copy(dst_hbm.at[pl.ds((c * NS + s) * KCNT, KCNT)], dst_v)
    plsc.subcore_barrier()

    def body(j, _):
        pltpu.sync_copy(buf_v, acc_sh.at[dst_v.at[j]], add=True)
        return 0
    lax.fori_loop(0, KCNT, body, 0)
    plsc.subcore_barrier()

    for r in range(ROWS_PER_TILE // CHUNK):
        pltpu.sync_copy(acc_sh.at[pl.ds(row0 + r * CHUNK, CHUNK)], buf_v)
        pltpu.sync_copy(buf_v, out_hbm.at[c, pl.ds(row0 + r * CHUNK, CHUNK)])


@functools.partial(
    pl.kernel,
    mesh=_MESH,
    out_type=jax.ShapeDtypeStruct((NC, N_ACC, D), jnp.float32),
    scratch_types=[
        pltpu.VMEM((KCNT, CHUNK), jnp.int32),
        pltpu.VMEM((KCNT, CHUNK), jnp.int32),
        pltpu.VMEM((CHUNK, D), jnp.float32),
        pltpu.VMEM_SHARED((N_ACC, D), jnp.float32),
        pltpu.SemaphoreType.DMA,
    ],
)
def _sc_scatter(y_hbm, src_hbm, dst_hbm, out_hbm, src_v, dst_v, rows_v,
                acc_sh, sem):
    c = lax.axis_index("c")
    s = lax.axis_index("s")
    row0 = s * ROWS_PER_TILE

    def zbody(i, _):
        for jj in range(D // 16):
            rows_v[i, pl.ds(jj * 16, 16)] = jnp.zeros((16,), jnp.float32)
        return 0
    lax.fori_loop(0, CHUNK, zbody, 0)
    for r in range(ROWS_PER_TILE // CHUNK):
        pltpu.sync_copy(rows_v, acc_sh.at[pl.ds(row0 + r * CHUNK, CHUNK)])
    tile_base = (c * NS + s) * KCNT
    pltpu.sync_copy(src_hbm.at[pl.ds(tile_base, KCNT)], src_v)
    pltpu.sync_copy(dst_hbm.at[pl.ds(tile_base, KCNT)], dst_v)
    plsc.subcore_barrier()

    def body(j, _):
        pltpu.async_copy(y_hbm.at[src_v.at[j]], rows_v, sem).wait()
        pltpu.sync_copy(rows_v, acc_sh.at[dst_v.at[j]], add=True)
        return 0
    lax.fori_loop(0, KCNT, body, 0)
    plsc.subcore_barrier()

    for r in range(ROWS_PER_TILE // CHUNK):
        pltpu.sync_copy(acc_sh.at[pl.ds(row0 + r * CHUNK, CHUNK)], rows_v)
        pltpu.sync_copy(rows_v, out_hbm.at[c, pl.ds(row0 + r * CHUNK, CHUNK)])


BN = 2000
GRID = N // BN


def _tc1_body(x_ref, w_ref, cnt_ref, y_ref, dinv_ref):
    deg = cnt_ref[0, :, 0:1] + cnt_ref[1, :, 0:1] + 1.0
    dinv = lax.rsqrt(deg)
    xw = jnp.dot(x_ref[...], w_ref[...], preferred_element_type=jnp.float32, precision=lax.Precision.HIGHEST)
    y_ref[...] = xw * dinv
    dinv_ref[...] = dinv


def _tc1(x, W1, cnt):
    return pl.pallas_call(
        _tc1_body,
        grid=(GRID,),
        in_specs=[
            pl.BlockSpec((BN, D), lambda i: (i, 0)),
            pl.BlockSpec((D, D), lambda i: (0, 0)),
            pl.BlockSpec((2, BN, D), lambda i: (0, i, 0)),
        ],
        out_specs=[
            pl.BlockSpec((BN, D), lambda i: (i, 0)),
            pl.BlockSpec((BN, 1), lambda i: (i, 0)),
        ],
        out_shape=[
            jax.ShapeDtypeStruct((N, D), jnp.float32),
            jax.ShapeDtypeStruct((N, 1), jnp.float32),
        ],
    )(x, W1, cnt)


def _tc2_body(p_ref, y_ref, dinv_ref, b1_ref, w_ref, y2_ref):
    dinv = dinv_ref[...]
    h = dinv * (p_ref[0] + p_ref[1] + y_ref[...]) + b1_ref[...]
    h = jnp.maximum(h, 0.0)
    y2_ref[...] = jnp.dot(h, w_ref[...],
                          preferred_element_type=jnp.float32, precision=lax.Precision.HIGHEST) * dinv


def _tc2(p, y1, dinv, b1, W2):
    return pl.pallas_call(
        _tc2_body,
        grid=(GRID,),
        in_specs=[
            pl.BlockSpec((2, BN, D), lambda i: (0, i, 0)),
            pl.BlockSpec((BN, D), lambda i: (i, 0)),
            pl.BlockSpec((BN, 1), lambda i: (i, 0)),
            pl.BlockSpec((1, D), lambda i: (0, 0)),
            pl.BlockSpec((D, D), lambda i: (0, 0)),
        ],
        out_specs=pl.BlockSpec((BN, D), lambda i: (i, 0)),
        out_shape=jax.ShapeDtypeStruct((N, D), jnp.float32),
    )(p, y1, dinv, b1, W2)


def _tc3_body(p_ref, y_ref, dinv_ref, b2_ref, batch_ref, wh1_ref, bh1_ref,
              wh2_ref, bh2_ref, out_ref, sums, cnts):
    i = pl.program_id(0)

    @pl.when(i == 0)
    def _():
        sums[...] = jnp.zeros((G, D), jnp.float32)
        cnts[...] = jnp.zeros((G, 1), jnp.float32)

    h = dinv_ref[...] * (p_ref[0] + p_ref[1] + y_ref[...]) + b2_ref[...]
    seg = lax.broadcasted_iota(jnp.int32, (BN, G), 1)
    onehot = jnp.where(batch_ref[...] == seg, 1.0, 0.0)
    sums[...] += lax.dot_general(onehot, h, (((0,), (0,)), ((), ())),
                                 preferred_element_type=jnp.float32, precision=lax.Precision.HIGHEST)
    cnts[...] += lax.dot_general(onehot, jnp.ones((BN, 1), jnp.float32),
                                 (((0,), (0,)), ((), ())),
                                 preferred_element_type=jnp.float32, precision=lax.Precision.HIGHEST)

    @pl.when(i == GRID - 1)
    def _():
        pooled = sums[...] / jnp.maximum(cnts[...], 1.0)
        z = jnp.dot(pooled, wh1_ref[...],
                    preferred_element_type=jnp.float32, precision=lax.Precision.HIGHEST) + bh1_ref[...]
        z = jnp.maximum(z, 0.0)
        z = jnp.dot(z, wh2_ref[...],
                    preferred_element_type=jnp.float32, precision=lax.Precision.HIGHEST) + bh2_ref[...]
        out_ref[...] = jax.nn.sigmoid(z)


def _tc3(p, y2, dinv, b2, batch, Wh1, bh1, Wh2, bh2):
    return pl.pallas_call(
        _tc3_body,
        grid=(GRID,),
        in_specs=[
            pl.BlockSpec((2, BN, D), lambda i: (0, i, 0)),
            pl.BlockSpec((BN, D), lambda i: (i, 0)),
            pl.BlockSpec((BN, 1), lambda i: (i, 0)),
            pl.BlockSpec((1, D), lambda i: (0, 0)),
            pl.BlockSpec((BN, 1), lambda i: (i, 0)),
            pl.BlockSpec((D, 10), lambda i: (0, 0)),
            pl.BlockSpec((1, 10), lambda i: (0, 0)),
            pl.BlockSpec((10, 1), lambda i: (0, 0)),
            pl.BlockSpec((1, 1), lambda i: (0, 0)),
        ],
        out_specs=pl.BlockSpec((G, 1), lambda i: (0, 0)),
        out_shape=jax.ShapeDtypeStruct((G, 1), jnp.float32),
        scratch_shapes=[
            pltpu.VMEM((G, D), jnp.float32),
            pltpu.VMEM((G, 1), jnp.float32),
        ],
    )(p, y2, dinv, b2, batch, Wh1, bh1, Wh2, bh2)


def kernel(x, edge_index, edge_attr, batch_idx, W1, b1, W2, b2, Wh1, bh1,
           Wh2, bh2):
    src = edge_index[0].astype(jnp.int32)
    dst = edge_index[1].astype(jnp.int32)
    pad = E_PAD - E
    src_p = jnp.concatenate([src, jnp.zeros((pad,), jnp.int32)])
    dst_p = jnp.concatenate([dst, jnp.full((pad,), DUMMY, jnp.int32)])
    src_p = src_p.reshape(TOTCH, CHUNK)
    dst_p = dst_p.reshape(TOTCH, CHUNK)

    cnt_raw = _sc_count(dst_p)
    y1, dinv = _tc1(x, W1, cnt_raw)
    p1 = _sc_scatter(y1, src_p, dst_p)
    y2 = _tc2(p1, y1, dinv, b1.reshape(1, D), W2)
    p2 = _sc_scatter(y2, src_p, dst_p)
    out = _tc3(p2, y2, dinv, b2.reshape(1, D),
               batch_idx.astype(jnp.int32).reshape(N, 1),
               Wh1, bh1.reshape(1, 10), Wh2, bh2.reshape(1, 1))
    return out


# speedup vs baseline: 2.5154x; 2.5154x over previous
import functools

import jax
import jax.numpy as jnp
from jax import lax
from jax.experimental import pallas as pl
from jax.experimental.pallas import tpu as pltpu
from jax.experimental.pallas import tpu_sc as plsc

N = 10000
E = 320000
D = 128
G = 64

NC = 2
NS = 16
CHUNK = 128
KCNT = 80
TOTCH = NC * NS * KCNT
E_PAD = TOTCH * CHUNK
KS = 40
ROWS_PER_TILE = 5 * CHUNK
N_ACC = NS * ROWS_PER_TILE
DUMMY = N

_MESH = plsc.VectorSubcoreMesh(core_axis_name="c", subcore_axis_name="s")


@functools.partial(
    pl.kernel,
    mesh=_MESH,
    out_type=jax.ShapeDtypeStruct((NC, N_ACC, D), jnp.float32),
    scratch_types=[
        pltpu.VMEM((KCNT, CHUNK), jnp.int32),
        pltpu.VMEM((CHUNK, D), jnp.float32),
        pltpu.VMEM_SHARED((N_ACC, D), jnp.float32),
    ],
)
def _sc_count(dst_hbm, out_hbm, dst_v, buf_v, acc_sh):
    c = lax.axis_index("c")
    s = lax.axis_index("s")
    row0 = s * ROWS_PER_TILE

    def fill(val):
        def body(i, _):
            for jj in range(D // 16):
                buf_v[i, pl.ds(jj * 16, 16)] = jnp.full((16,), val, jnp.float32)
            return 0
        lax.fori_loop(0, CHUNK, body, 0)

    fill(0.0)
    for r in range(ROWS_PER_TILE // CHUNK):
        pltpu.sync_copy(buf_v, acc_sh.at[pl.ds(row0 + r * CHUNK, CHUNK)])
    fill(1.0)
    pltpu.sync_copy(dst_hbm.at[pl.ds((c * NS + s) * KCNT, KCNT)], dst_v)
    plsc.subcore_barrier()

    def body(j, _):
        pltpu.sync_copy(buf_v, acc_sh.at[dst_v.at[j]], add=True)
        return 0
    lax.fori_loop(0, KCNT, body, 0)
    plsc.subcore_barrier()

    for r in range(ROWS_PER_TILE // CHUNK):
        pltpu.sync_copy(acc_sh.at[pl.ds(row0 + r * CHUNK, CHUNK)], buf_v)
        pltpu.sync_copy(buf_v, out_hbm.at[c, pl.ds(row0 + r * CHUNK, CHUNK)])


@functools.partial(
    pl.kernel,
    mesh=_MESH,
    out_type=jax.ShapeDtypeStruct((NC, N_ACC, D), jnp.float32),
    scratch_types=[
        pltpu.VMEM((KCNT, CHUNK), jnp.int32),
        pltpu.VMEM((KCNT, CHUNK), jnp.int32),
        pltpu.VMEM((CHUNK, D), jnp.float32),
        pltpu.VMEM_SHARED((N_ACC, D), jnp.float32),
        pltpu.SemaphoreType.DMA,
    ],
)
def _sc_scatter(y_hbm, src_hbm, dst_hbm, out_hbm, src_v, dst_v, rows_v,
                acc_sh, sem):
    c = lax.axis_index("c")
    s = lax.axis_index("s")
    row0 = s * ROWS_PER_TILE

    def zbody(i, _):
        for jj in range(D // 16):
            rows_v[i, pl.ds(jj * 16, 16)] = jnp.zeros((16,), jnp.float32)
        return 0
    lax.fori_loop(0, CHUNK, zbody, 0)
    for r in range(ROWS_PER_TILE // CHUNK):
        pltpu.sync_copy(rows_v, acc_sh.at[pl.ds(row0 + r * CHUNK, CHUNK)])
    tile_base = (c * NS + s) * KCNT
    pltpu.sync_copy(src_hbm.at[pl.ds(tile_base, KCNT)], src_v)
    pltpu.sync_copy(dst_hbm.at[pl.ds(tile_base, KCNT)], dst_v)
    plsc.subcore_barrier()

    def body(j, _):
        pltpu.async_copy(y_hbm.at[src_v.at[j]], rows_v, sem).wait()
        pltpu.sync_copy(rows_v, acc_sh.at[dst_v.at[j]], add=True)
        return 0
    lax.fori_loop(0, KCNT, body, 0)
    plsc.subcore_barrier()

    for r in range(ROWS_PER_TILE // CHUNK):
        pltpu.sync_copy(acc_sh.at[pl.ds(row0 + r * CHUNK, CHUNK)], rows_v)
        pltpu.sync_copy(rows_v, out_hbm.at[c, pl.ds(row0 + r * CHUNK, CHUNK)])


BN = 2000
GRID = N // BN


def _tc1_body(x_ref, w_ref, cnt_ref, y_ref, dinv_ref):
    deg = cnt_ref[0, :, 0:1] + cnt_ref[1, :, 0:1] + 1.0
    dinv = lax.rsqrt(deg)
    xw = jnp.dot(x_ref[...], w_ref[...], preferred_element_type=jnp.float32, precision=lax.Precision.HIGHEST)
    y_ref[...] = xw * dinv
    dinv_ref[...] = dinv


def _tc1(x, W1, cnt):
    return pl.pallas_call(
        _tc1_body,
        grid=(GRID,),
        in_specs=[
            pl.BlockSpec((BN, D), lambda i: (i, 0)),
            pl.BlockSpec((D, D), lambda i: (0, 0)),
            pl.BlockSpec((2, BN, D), lambda i: (0, i, 0)),
        ],
        out_specs=[
            pl.BlockSpec((BN, D), lambda i: (i, 0)),
            pl.BlockSpec((BN, 1), lambda i: (i, 0)),
        ],
        out_shape=[
            jax.ShapeDtypeStruct((N, D), jnp.float32),
            jax.ShapeDtypeStruct((N, 1), jnp.float32),
        ],
    )(x, W1, cnt)


def _tc2_body(p_ref, y_ref, dinv_ref, b1_ref, w_ref, y2_ref):
    dinv = dinv_ref[...]
    h = dinv * (p_ref[0] + p_ref[1] + y_ref[...]) + b1_ref[...]
    h = jnp.maximum(h, 0.0)
    y2_ref[...] = jnp.dot(h, w_ref[...],
                          preferred_element_type=jnp.float32, precision=lax.Precision.HIGHEST) * dinv


def _tc2(p, y1, dinv, b1, W2):
    return pl.pallas_call(
        _tc2_body,
        grid=(GRID,),
        in_specs=[
            pl.BlockSpec((2, BN, D), lambda i: (0, i, 0)),
            pl.BlockSpec((BN, D), lambda i: (i, 0)),
            pl.BlockSpec((BN, 1), lambda i: (i, 0)),
            pl.BlockSpec((1, D), lambda i: (0, 0)),
            pl.BlockSpec((D, D), lambda i: (0, 0)),
        ],
        out_specs=pl.BlockSpec((BN, D), lambda i: (i, 0)),
        out_shape=jax.ShapeDtypeStruct((N, D), jnp.float32),
    )(p, y1, dinv, b1, W2)


def _tc3_body(p_ref, y_ref, dinv_ref, b2_ref, batch_ref, wh1_ref, bh1_ref,
              wh2_ref, bh2_ref, out_ref, sums, cnts):
    i = pl.program_id(0)

    @pl.when(i == 0)
    def _():
        sums[...] = jnp.zeros((G, D), jnp.float32)
        cnts[...] = jnp.zeros((G, 1), jnp.float32)

    h = dinv_ref[...] * (p_ref[0] + p_ref[1] + y_ref[...]) + b2_ref[...]
    seg = lax.broadcasted_iota(jnp.int32, (BN, G), 1)
    onehot = jnp.where(batch_ref[...] == seg, 1.0, 0.0)
    sums[...] += lax.dot_general(onehot, h, (((0,), (0,)), ((), ())),
                                 preferred_element_type=jnp.float32, precision=lax.Precision.HIGHEST)
    cnts[...] += lax.dot_general(onehot, jnp.ones((BN, 1), jnp.float32),
                                 (((0,), (0,)), ((), ())),
                                 preferred_element_type=jnp.float32, precision=lax.Precision.HIGHEST)

    @pl.when(i == GRID - 1)
    def _():
        pooled = sums[...] / jnp.maximum(cnts[...], 1.0)
        z = jnp.dot(pooled, wh1_ref[...],
                    preferred_element_type=jnp.float32, precision=lax.Precision.HIGHEST) + bh1_ref[...]
        z = jnp.maximum(z, 0.0)
        z = jnp.dot(z, wh2_ref[...],
                    preferred_element_type=jnp.float32, precision=lax.Precision.HIGHEST) + bh2_ref[...]
        out_ref[...] = jax.nn.sigmoid(z)


def _tc3(p, y2, dinv, b2, batch, Wh1, bh1, Wh2, bh2):
    return pl.pallas_call(
        _tc3_body,
        grid=(GRID,),
        in_specs=[
            pl.BlockSpec((2, BN, D), lambda i: (0, i, 0)),
            pl.BlockSpec((BN, D), lambda i: (i, 0)),
            pl.BlockSpec((BN, 1), lambda i: (i, 0)),
            pl.BlockSpec((1, D), lambda i: (0, 0)),
            pl.BlockSpec((BN, 1), lambda i: (i, 0)),
            pl.BlockSpec((D, 10), lambda i: (0, 0)),
            pl.BlockSpec((1, 10), lambda i: (0, 0)),
            pl.BlockSpec((10, 1), lambda i: (0, 0)),
            pl.BlockSpec((1, 1), lambda i: (0, 0)),
        ],
        out_specs=pl.BlockSpec((G, 1), lambda i: (0, 0)),
        out_shape=jax.ShapeDtypeStruct((G, 1), jnp.float32),
        scratch_shapes=[
            pltpu.VMEM((G, D), jnp.float32),
            pltpu.VMEM((G, 1), jnp.float32),
        ],
    )(p, y2, dinv, b2, batch, Wh1, bh1, Wh2, bh2)


def kernel(x, edge_index, edge_attr, batch_idx, W1, b1, W2, b2, Wh1, bh1,
           Wh2, bh2):
    src = edge_index[0].astype(jnp.int32)
    dst = edge_index[1].astype(jnp.int32)
    pad = E_PAD - E
    idxp = jnp.arange(pad, dtype=jnp.int32)
    src_p = jnp.concatenate([src, idxp % N])
    dst_p = jnp.concatenate([dst, DUMMY + idxp % (N_ACC - N)])
    src_p = src_p.reshape(TOTCH, CHUNK)
    dst_p = dst_p.reshape(TOTCH, CHUNK)

    cnt_raw = _sc_count(dst_p)
    y1, dinv = _tc1(x, W1, cnt_raw)
    p1 = _sc_scatter(y1, src_p, dst_p)
    y2 = _tc2(p1, y1, dinv, b1.reshape(1, D), W2)
    p2 = _sc_scatter(y2, src_p, dst_p)
    out = _tc3(p2, y2, dinv, b2.reshape(1, D),
               batch_idx.astype(jnp.int32).reshape(N, 1),
               Wh1, bh1.reshape(1, 10), Wh2, bh2.reshape(1, 1))
    return out


# speedup vs baseline: 3.4380x; 1.3668x over previous
import functools

import jax
import jax.numpy as jnp
from jax import lax
from jax.experimental import pallas as pl
from jax.experimental.pallas import tpu as pltpu
from jax.experimental.pallas import tpu_sc as plsc

N = 10000
E = 320000
D = 128
G = 64

NC = 2
NS = 16
CHUNK = 128
KCNT = 80
TOTCH = NC * NS * KCNT
E_PAD = TOTCH * CHUNK
KS = 40
ROWS_PER_TILE = 5 * CHUNK
N_ACC = NS * ROWS_PER_TILE
DUMMY = N

_MESH = plsc.VectorSubcoreMesh(core_axis_name="c", subcore_axis_name="s")


@functools.partial(
    pl.kernel,
    mesh=_MESH,
    out_type=jax.ShapeDtypeStruct((NC, N_ACC, D), jnp.float32),
    scratch_types=[
        pltpu.VMEM((KCNT, CHUNK), jnp.int32),
        pltpu.VMEM((CHUNK, D), jnp.float32),
        pltpu.VMEM_SHARED((N_ACC, D), jnp.float32),
    ],
)
def _sc_count(dst_hbm, out_hbm, dst_v, buf_v, acc_sh):
    c = lax.axis_index("c")
    s = lax.axis_index("s")
    row0 = s * ROWS_PER_TILE

    def fill(val):
        def body(i, _):
            for jj in range(D // 16):
                buf_v[i, pl.ds(jj * 16, 16)] = jnp.full((16,), val, jnp.float32)
            return 0
        lax.fori_loop(0, CHUNK, body, 0)

    fill(0.0)
    for r in range(ROWS_PER_TILE // CHUNK):
        pltpu.sync_copy(buf_v, acc_sh.at[pl.ds(row0 + r * CHUNK, CHUNK)])
    fill(1.0)
    pltpu.sync_copy(dst_hbm.at[pl.ds((c * NS + s) * KCNT, KCNT)], dst_v)
    plsc.subcore_barrier()

    def body(j, _):
        pltpu.sync_copy(buf_v, acc_sh.at[dst_v.at[j]], add=True)
        return 0
    lax.fori_loop(0, KCNT, body, 0)
    plsc.subcore_barrier()

    for r in range(ROWS_PER_TILE // CHUNK):
        pltpu.sync_copy(acc_sh.at[pl.ds(row0 + r * CHUNK, CHUNK)], buf_v)
        pltpu.sync_copy(buf_v, out_hbm.at[c, pl.ds(row0 + r * CHUNK, CHUNK)])


@functools.partial(
    pl.kernel,
    mesh=_MESH,
    out_type=jax.ShapeDtypeStruct((NC, N_ACC, D), jnp.float32),
    scratch_types=[
        pltpu.VMEM((KS, CHUNK), jnp.int32),
        pltpu.VMEM((KS, CHUNK), jnp.int32),
        pltpu.VMEM((CHUNK, D), jnp.float32),
        pltpu.VMEM((CHUNK, D), jnp.float32),
        pltpu.VMEM_SHARED((N_ACC, D), jnp.float32),
        pltpu.SemaphoreType.DMA,
        pltpu.SemaphoreType.DMA,
    ],
)
def _sc_scatter(y_hbm, src_hbm, dst_hbm, out_hbm, src_v, dst_v, r0, r1,
                acc_sh, s0, s1):
    c = lax.axis_index("c")
    s = lax.axis_index("s")
    row0 = s * ROWS_PER_TILE

    def zbody(i, _):
        for jj in range(D // 16):
            r0[i, pl.ds(jj * 16, 16)] = jnp.zeros((16,), jnp.float32)
        return 0
    lax.fori_loop(0, CHUNK, zbody, 0)
    for r in range(ROWS_PER_TILE // CHUNK):
        pltpu.sync_copy(r0, acc_sh.at[pl.ds(row0 + r * CHUNK, CHUNK)])
    plsc.subcore_barrier()

    tile_base = (c * NS + s) * KCNT
    for seg in range(KCNT // KS):
        base = tile_base + seg * KS
        pltpu.sync_copy(src_hbm.at[pl.ds(base, KS)], src_v)
        pltpu.sync_copy(dst_hbm.at[pl.ds(base, KS)], dst_v)
        pltpu.async_copy(y_hbm.at[src_v.at[0]], r0, s0)

        def body(g, _):
            a = g * 2
            pltpu.async_copy(y_hbm.at[src_v.at[a + 1]], r1, s1)
            pltpu.make_async_copy(y_hbm.at[src_v.at[a]], r0, s0).wait()
            pltpu.sync_copy(r0, acc_sh.at[dst_v.at[a]], add=True)
            nxt = jnp.minimum(a + 2, KS - 1)
            pltpu.async_copy(y_hbm.at[src_v.at[nxt]], r0, s0)
            pltpu.make_async_copy(y_hbm.at[src_v.at[a + 1]], r1, s1).wait()
            pltpu.sync_copy(r1, acc_sh.at[dst_v.at[a + 1]], add=True)
            return 0
        lax.fori_loop(0, KS // 2, body, 0)
        pltpu.make_async_copy(y_hbm.at[src_v.at[KS - 1]], r0, s0).wait()
    plsc.subcore_barrier()

    for r in range(ROWS_PER_TILE // CHUNK):
        pltpu.sync_copy(acc_sh.at[pl.ds(row0 + r * CHUNK, CHUNK)], r0)
        pltpu.sync_copy(r0, out_hbm.at[c, pl.ds(row0 + r * CHUNK, CHUNK)])


BN = 2000
GRID = N // BN


def _tc1_body(x_ref, w_ref, cnt_ref, y_ref, dinv_ref):
    deg = cnt_ref[0, :, 0:1] + cnt_ref[1, :, 0:1] + 1.0
    dinv = lax.rsqrt(deg)
    xw = jnp.dot(x_ref[...], w_ref[...], preferred_element_type=jnp.float32, precision=lax.Precision.HIGHEST)
    y_ref[...] = xw * dinv
    dinv_ref[...] = dinv


def _tc1(x, W1, cnt):
    return pl.pallas_call(
        _tc1_body,
        grid=(GRID,),
        in_specs=[
            pl.BlockSpec((BN, D), lambda i: (i, 0)),
            pl.BlockSpec((D, D), lambda i: (0, 0)),
            pl.BlockSpec((2, BN, D), lambda i: (0, i, 0)),
        ],
        out_specs=[
            pl.BlockSpec((BN, D), lambda i: (i, 0)),
            pl.BlockSpec((BN, 1), lambda i: (i, 0)),
        ],
        out_shape=[
            jax.ShapeDtypeStruct((N, D), jnp.float32),
            jax.ShapeDtypeStruct((N, 1), jnp.float32),
        ],
    )(x, W1, cnt)


def _tc2_body(p_ref, y_ref, dinv_ref, b1_ref, w_ref, y2_ref):
    dinv = dinv_ref[...]
    h = dinv * (p_ref[0] + p_ref[1] + y_ref[...]) + b1_ref[...]
    h = jnp.maximum(h, 0.0)
    y2_ref[...] = jnp.dot(h, w_ref[...],
                          preferred_element_type=jnp.float32, precision=lax.Precision.HIGHEST) * dinv


def _tc2(p, y1, dinv, b1, W2):
    return pl.pallas_call(
        _tc2_body,
        grid=(GRID,),
        in_specs=[
            pl.BlockSpec((2, BN, D), lambda i: (0, i, 0)),
            pl.BlockSpec((BN, D), lambda i: (i, 0)),
            pl.BlockSpec((BN, 1), lambda i: (i, 0)),
            pl.BlockSpec((1, D), lambda i: (0, 0)),
            pl.BlockSpec((D, D), lambda i: (0, 0)),
        ],
        out_specs=pl.BlockSpec((BN, D), lambda i: (i, 0)),
        out_shape=jax.ShapeDtypeStruct((N, D), jnp.float32),
    )(p, y1, dinv, b1, W2)


def _tc3_body(p_ref, y_ref, dinv_ref, b2_ref, batch_ref, wh1_ref, bh1_ref,
              wh2_ref, bh2_ref, out_ref, sums, cnts):
    i = pl.program_id(0)

    @pl.when(i == 0)
    def _():
        sums[...] = jnp.zeros((G, D), jnp.float32)
        cnts[...] = jnp.zeros((G, 1), jnp.float32)

    h = dinv_ref[...] * (p_ref[0] + p_ref[1] + y_ref[...]) + b2_ref[...]
    seg = lax.broadcasted_iota(jnp.int32, (BN, G), 1)
    onehot = jnp.where(batch_ref[...] == seg, 1.0, 0.0)
    sums[...] += lax.dot_general(onehot, h, (((0,), (0,)), ((), ())),
                                 preferred_element_type=jnp.float32, precision=lax.Precision.HIGHEST)
    cnts[...] += lax.dot_general(onehot, jnp.ones((BN, 1), jnp.float32),
                                 (((0,), (0,)), ((), ())),
                                 preferred_element_type=jnp.float32, precision=lax.Precision.HIGHEST)

    @pl.when(i == GRID - 1)
    def _():
        pooled = sums[...] / jnp.maximum(cnts[...], 1.0)
        z = jnp.dot(pooled, wh1_ref[...],
                    preferred_element_type=jnp.float32, precision=lax.Precision.HIGHEST) + bh1_ref[...]
        z = jnp.maximum(z, 0.0)
        z = jnp.dot(z, wh2_ref[...],
                    preferred_element_type=jnp.float32, precision=lax.Precision.HIGHEST) + bh2_ref[...]
        out_ref[...] = jax.nn.sigmoid(z)


def _tc3(p, y2, dinv, b2, batch, Wh1, bh1, Wh2, bh2):
    return pl.pallas_call(
        _tc3_body,
        grid=(GRID,),
        in_specs=[
            pl.BlockSpec((2, BN, D), lambda i: (0, i, 0)),
            pl.BlockSpec((BN, D), lambda i: (i, 0)),
            pl.BlockSpec((BN, 1), lambda i: (i, 0)),
            pl.BlockSpec((1, D), lambda i: (0, 0)),
            pl.BlockSpec((BN, 1), lambda i: (i, 0)),
            pl.BlockSpec((D, 10), lambda i: (0, 0)),
            pl.BlockSpec((1, 10), lambda i: (0, 0)),
            pl.BlockSpec((10, 1), lambda i: (0, 0)),
            pl.BlockSpec((1, 1), lambda i: (0, 0)),
        ],
        out_specs=pl.BlockSpec((G, 1), lambda i: (0, 0)),
        out_shape=jax.ShapeDtypeStruct((G, 1), jnp.float32),
        scratch_shapes=[
            pltpu.VMEM((G, D), jnp.float32),
            pltpu.VMEM((G, 1), jnp.float32),
        ],
    )(p, y2, dinv, b2, batch, Wh1, bh1, Wh2, bh2)


def kernel(x, edge_index, edge_attr, batch_idx, W1, b1, W2, b2, Wh1, bh1,
           Wh2, bh2):
    src = edge_index[0].astype(jnp.int32)
    dst = edge_index[1].astype(jnp.int32)
    pad = E_PAD - E
    idxp = jnp.arange(pad, dtype=jnp.int32)
    src_p = jnp.concatenate([src, idxp % N])
    dst_p = jnp.concatenate([dst, DUMMY + idxp % (N_ACC - N)])
    src_p = src_p.reshape(TOTCH, CHUNK)
    dst_p = dst_p.reshape(TOTCH, CHUNK)

    cnt_raw = _sc_count(dst_p)
    y1, dinv = _tc1(x, W1, cnt_raw)
    p1 = _sc_scatter(y1, src_p, dst_p)
    y2 = _tc2(p1, y1, dinv, b1.reshape(1, D), W2)
    p2 = _sc_scatter(y2, src_p, dst_p)
    out = _tc3(p2, y2, dinv, b2.reshape(1, D),
               batch_idx.astype(jnp.int32).reshape(N, 1),
               Wh1, bh1.reshape(1, 10), Wh2, bh2.reshape(1, 1))
    return out
